# baseline (speedup 1.0000x reference)
import functools

import jax
import jax.numpy as jnp
from jax import lax
from jax.experimental import pallas as pl
from jax.experimental.pallas import tpu as pltpu
from jax.experimental.pallas import tpu_sc as plsc

HEADS = 16
HEAD_DIM = 128
OUT_DIM = 1024
BATCH = 4
KTOK = 512
TTOK = 4096
ROWS = HEADS * KTOK

NTILES = 32
SW = 16
CH = 512
NCH = ROWS // CH
TRL = CH // 8
UNROLL = 16


def _mm_body(y_ref, w_ref, x_ref):
    x = jnp.dot(y_ref[0, 0], w_ref[0], preferred_element_type=jnp.float32)
    x_ref[0] = x.reshape(KTOK // 8, 8, 8, 128)


def _tc_einsum(Y, W):
    return pl.pallas_call(
        _mm_body,
        grid=(BATCH, HEADS),
        in_specs=[
            pl.BlockSpec((1, 1, KTOK, HEAD_DIM), lambda b, e: (b, e, 0, 0)),
            pl.BlockSpec((1, HEAD_DIM, OUT_DIM), lambda b, e: (e, 0, 0)),
        ],
        out_specs=pl.BlockSpec((1, KTOK // 8, 8, 8, 128),
                               lambda b, e: (b, e, 0, 0, 0)),
        out_shape=jax.ShapeDtypeStruct((BATCH, ROWS // 8, 8, 8, 128),
                                       jnp.float32),
    )(Y, W)


@functools.partial(
    pl.kernel,
    out_type=jax.ShapeDtypeStruct((BATCH, TTOK // 8, 64, 128), jnp.float32),
    mesh=plsc.VectorSubcoreMesh(core_axis_name="c", subcore_axis_name="s"),
    compiler_params=pltpu.CompilerParams(use_tc_tiling_on_sc=False,
                                         needs_layout_passes=False),
    scratch_types=[
        pltpu.VMEM((TTOK // 8, 8, SW), jnp.float32),
        pltpu.VMEM((2, TRL, 8, 1, SW), jnp.float32),
        pltpu.VMEM((2, CH), jnp.int32),
        pltpu.SemaphoreType.DMA,
        pltpu.SemaphoreType.DMA,
        pltpu.SemaphoreType.DMA,
    ],
)
def _sc_scatter(x_hbm, ind_hbm, out_hbm, acc, rows2, idxv2, semA, semB, semF):
    c = lax.axis_index("c")
    s = lax.axis_index("s")
    wid = c * 16 + s
    b = wid // 8
    g = wid % 8

    zero16 = jnp.zeros((16,), jnp.float32)
    lanes = lax.iota(jnp.int32, 16)
    sels = [jnp.full((16,), j, jnp.int32) for j in range(UNROLL)]

    def _start(q, cin, buf, sem):
        pltpu.make_async_copy(
            ind_hbm.at[pl.ds(b * ROWS + q * CH, CH)], idxv2.at[buf],
            sem).start()
        pltpu.make_async_copy(
            x_hbm.at[b, pl.ds(q * TRL, TRL), pl.ds(0, 8), pl.ds(g, 1),
                     pl.ds(cin, SW)],
            rows2.at[buf], sem).start()

    def _wait(q, cin, buf, sem):
        pltpu.make_async_copy(
            ind_hbm.at[pl.ds(b * ROWS + q * CH, CH)], idxv2.at[buf],
            sem).wait()
        pltpu.make_async_copy(
            x_hbm.at[b, pl.ds(q * TRL, TRL), pl.ds(0, 8), pl.ds(g, 1),
                     pl.ds(cin, SW)],
            rows2.at[buf], sem).wait()

    def _compute(buf):
        rows = rows2.at[buf]
        idxs = idxv2.at[buf]

        @plsc.parallel_loop(0, CH // UNROLL, unroll=4)
        def _grp(k):
            base = k * UNROLL
            idx16 = idxs[pl.ds(base, UNROLL)]
            trl2 = 2 * k
            for j in range(UNROLL):
                rowv = idx16.at[sels[j]].get(mode="promise_in_bounds")
                trv = lax.shift_right_logical(rowv, 3)
                rv = lax.bitwise_and(rowv, 7)
                plsc.addupdate_scatter(
                    acc, [trv, rv, lanes],
                    rows[trl2 + j // 8, j % 8, 0, pl.ds(0, 16)])

    for t in range(8):
        cin = t * SW

        @plsc.parallel_loop(0, TTOK // 8, unroll=2)
        def _za(i):
            for j in range(8):
                acc[i, j, pl.ds(0, 16)] = zero16

        _start(0, cin, 0, semA)

        def _pair(p, carry):
            q0 = 2 * p

            @pl.when(q0 + 1 < NCH)
            def _():
                _start(q0 + 1, cin, 1, semB)

            _wait(q0, cin, 0, semA)
            _compute(0)

            @pl.when(q0 + 2 < NCH)
            def _():
                _start(q0 + 2, cin, 0, semA)

            @pl.when(q0 + 1 < NCH)
            def _():
                _wait(q0 + 1, cin, 1, semB)
                _compute(1)

            return carry

        lax.fori_loop(0, (NCH + 1) // 2, _pair, 0)

        dst = out_hbm.at[b, pl.ds(0, TTOK // 8), pl.ds(g * 8, 8),
                         pl.ds(cin, SW)]
        pltpu.make_async_copy(acc, dst, semF).start()
        pltpu.make_async_copy(acc, dst, semF).wait()


def kernel(Y, Ind, T, W):
    X = _tc_einsum(Y.astype(jnp.bfloat16), W.astype(jnp.bfloat16))
    idx = jnp.mod(Ind.reshape(BATCH * ROWS).astype(jnp.int32),
                  jnp.asarray(T, jnp.int32))
    out5 = _sc_scatter(X, idx)
    out = (out5.reshape(BATCH, TTOK // 8, 8, 8, 128)
           .transpose(0, 1, 3, 2, 4)
           .reshape(BATCH, TTOK, OUT_DIM))
    return out


# speedup vs baseline: 1.1402x; 1.1402x over previous
import functools

import jax
import jax.numpy as jnp
from jax import lax
from jax.experimental import pallas as pl
from jax.experimental.pallas import tpu as pltpu
from jax.experimental.pallas import tpu_sc as plsc

HEADS = 16
HEAD_DIM = 128
OUT_DIM = 1024
BATCH = 4
KTOK = 512
TTOK = 4096
ROWS = HEADS * KTOK

NTILES = 32
SW = 16
CH = 1024
NCH = ROWS // CH
TRL = CH // 8
UNROLL = 16


def _mm_body(y_ref, w_ref, x_ref):
    x = jnp.dot(y_ref[0, 0], w_ref[0], preferred_element_type=jnp.float32)
    x_ref[0] = x.reshape(KTOK // 8, 8, 8, 128)


def _tc_einsum(Y, W):
    return pl.pallas_call(
        _mm_body,
        grid=(BATCH, HEADS),
        in_specs=[
            pl.BlockSpec((1, 1, KTOK, HEAD_DIM), lambda b, e: (b, e, 0, 0)),
            pl.BlockSpec((1, HEAD_DIM, OUT_DIM), lambda b, e: (e, 0, 0)),
        ],
        out_specs=pl.BlockSpec((1, KTOK // 8, 8, 8, 128),
                               lambda b, e: (b, e, 0, 0, 0)),
        out_shape=jax.ShapeDtypeStruct((BATCH, ROWS // 8, 8, 8, 128),
                                       jnp.float32),
    )(Y, W)


@functools.partial(
    pl.kernel,
    out_type=jax.ShapeDtypeStruct((BATCH, TTOK // 8, 64, 128), jnp.float32),
    mesh=plsc.VectorSubcoreMesh(core_axis_name="c", subcore_axis_name="s"),
    compiler_params=pltpu.CompilerParams(use_tc_tiling_on_sc=False,
                                         needs_layout_passes=False),
    scratch_types=[
        pltpu.VMEM((TTOK // 8, 8, SW), jnp.float32),
        pltpu.VMEM((2, TRL, 8, 1, SW), jnp.float32),
        pltpu.VMEM((2, CH), jnp.int32),
        pltpu.SemaphoreType.DMA,
        pltpu.SemaphoreType.DMA,
        pltpu.SemaphoreType.DMA,
    ],
)
def _sc_scatter(x_hbm, ind_hbm, out_hbm, acc, rows2, idxv2, semA, semB, semF):
    c = lax.axis_index("c")
    s = lax.axis_index("s")
    wid = c * 16 + s
    b = wid // 8
    g = wid % 8

    zero16 = jnp.zeros((16,), jnp.float32)
    lanes = lax.iota(jnp.int32, 16)
    sels = [jnp.full((16,), j, jnp.int32) for j in range(UNROLL)]

    def _start(q, cin, buf, sem):
        pltpu.make_async_copy(
            ind_hbm.at[pl.ds(b * ROWS + q * CH, CH)], idxv2.at[buf],
            sem).start()
        pltpu.make_async_copy(
            x_hbm.at[b, pl.ds(q * TRL, TRL), pl.ds(0, 8), pl.ds(g, 1),
                     pl.ds(cin, SW)],
            rows2.at[buf], sem).start()

    def _wait(q, cin, buf, sem):
        pltpu.make_async_copy(
            ind_hbm.at[pl.ds(b * ROWS + q * CH, CH)], idxv2.at[buf],
            sem).wait()
        pltpu.make_async_copy(
            x_hbm.at[b, pl.ds(q * TRL, TRL), pl.ds(0, 8), pl.ds(g, 1),
                     pl.ds(cin, SW)],
            rows2.at[buf], sem).wait()

    def _compute(buf):
        rows = rows2.at[buf]
        idxs = idxv2.at[buf]

        @plsc.parallel_loop(0, CH // UNROLL, unroll=4)
        def _grp(k):
            base = k * UNROLL
            idx16 = idxs[pl.ds(base, UNROLL)]
            trl2 = 2 * k
            for j in range(UNROLL):
                rowv = idx16.at[sels[j]].get(mode="promise_in_bounds")
                trv = lax.shift_right_logical(rowv, 3)
                rv = lax.bitwise_and(rowv, 7)
                plsc.addupdate_scatter(
                    acc, [trv, rv, lanes],
                    rows[trl2 + j // 8, j % 8, 0, pl.ds(0, 16)])

    prev_dst = None
    for t in range(8):
        cin = t * SW

        _start(0, cin, 0, semA)
        if prev_dst is not None:
            pltpu.make_async_copy(acc, prev_dst, semF).wait()

        @plsc.parallel_loop(0, TTOK // 8, unroll=2)
        def _za(i):
            for j in range(8):
                acc[i, j, pl.ds(0, 16)] = zero16

        def _pair(p, carry):
            q0 = 2 * p

            @pl.when(q0 + 1 < NCH)
            def _():
                _start(q0 + 1, cin, 1, semB)

            _wait(q0, cin, 0, semA)
            _compute(0)

            @pl.when(q0 + 2 < NCH)
            def _():
                _start(q0 + 2, cin, 0, semA)

            @pl.when(q0 + 1 < NCH)
            def _():
                _wait(q0 + 1, cin, 1, semB)
                _compute(1)

            return carry

        lax.fori_loop(0, (NCH + 1) // 2, _pair, 0)

        dst = out_hbm.at[b, pl.ds(0, TTOK // 8), pl.ds(g * 8, 8),
                         pl.ds(cin, SW)]
        pltpu.make_async_copy(acc, dst, semF).start()
        prev_dst = dst
    pltpu.make_async_copy(acc, prev_dst, semF).wait()


def kernel(Y, Ind, T, W):
    X = _tc_einsum(Y, W)
    idx = jnp.mod(Ind.reshape(BATCH * ROWS).astype(jnp.int32),
                  jnp.asarray(T, jnp.int32))
    out5 = _sc_scatter(X, idx)
    out = (out5.reshape(BATCH, TTOK // 8, 8, 8, 128)
           .transpose(0, 1, 3, 2, 4)
           .reshape(BATCH, TTOK, OUT_DIM))
    return out
